# Initial kernel scaffold; baseline (speedup 1.0000x reference)
import jax
import jax.numpy as jnp
from jax.experimental import pallas as pl


def kernel(x_cat, tables, W, b):
    raise NotImplementedError("write your pallas kernel here")


# speedup vs baseline: 1.5558x; 1.5558x over previous
import functools

import jax
import jax.numpy as jnp
from jax import lax
from jax.experimental import pallas as pl
from jax.experimental.pallas import tpu as pltpu
from jax.experimental.pallas import tpu_sc as plsc

_B = 4096
_S = 50
_NF = 26
_CARD = 100000
_EDIM = 32
_DMODEL = 128

_T = _B * _S
_R = _T * _NF
_LPB = 128
_NBLK = _R // _LPB
_CHUNK = 10


def _sc_gather(flat_tables, gidx):
    info = plsc.get_sparse_core_info()
    nw = info.num_cores * info.num_subcores
    blocks_per_w = _NBLK // nw
    iters = blocks_per_w // _CHUNK
    crows = _CHUNK * _LPB

    mesh = plsc.VectorSubcoreMesh(core_axis_name="c", subcore_axis_name="s")

    @functools.partial(
        pl.kernel,
        mesh=mesh,
        compiler_params=pltpu.CompilerParams(use_tc_tiling_on_sc=False),
        out_type=jax.ShapeDtypeStruct((_R, _EDIM), jnp.float32),
        scratch_types=[
            pltpu.VMEM((crows,), jnp.int32),
            pltpu.VMEM((crows, _EDIM), jnp.float32),
            pltpu.SemaphoreType.DMA,
        ],
    )
    def k(tab_hbm, gidx_hbm, out_hbm, idx_v, rows_v, sem):
        wid = lax.axis_index("s") * info.num_cores + lax.axis_index("c")
        base = wid * blocks_per_w

        def body(it, carry):
            row0 = (base + it * _CHUNK) * _LPB
            pltpu.sync_copy(gidx_hbm.at[pl.ds(row0, crows)], idx_v)
            copies = []
            for j in range(_CHUNK):
                copies.append(
                    pltpu.async_copy(
                        tab_hbm.at[idx_v.at[pl.ds(j * _LPB, _LPB)]],
                        rows_v.at[pl.ds(j * _LPB, _LPB)],
                        sem,
                    )
                )
            for c in copies:
                c.wait()
            pltpu.sync_copy(rows_v, out_hbm.at[pl.ds(row0, crows)])
            return carry

        lax.fori_loop(0, iters, body, 0)

    return k(flat_tables, gidx)


def _tc_project(a, w, bias):
    tile = 1024
    grid = (_T // tile,)

    def mm(a_ref, w_ref, b_ref, o_ref):
        o_ref[...] = (
            jnp.dot(a_ref[...], w_ref[...], preferred_element_type=jnp.float32)
            + b_ref[...]
        )

    return pl.pallas_call(
        mm,
        grid=grid,
        in_specs=[
            pl.BlockSpec((tile, _NF * _EDIM), lambda i: (i, 0)),
            pl.BlockSpec((_NF * _EDIM, _DMODEL), lambda i: (0, 0)),
            pl.BlockSpec((1, _DMODEL), lambda i: (0, 0)),
        ],
        out_specs=pl.BlockSpec((tile, _DMODEL), lambda i: (i, 0)),
        out_shape=jax.ShapeDtypeStruct((_T, _DMODEL), jnp.float32),
    )(a, w, bias)


def kernel(x_cat, tables, W, b):
    flat_tables = tables.reshape(_NF * (_CARD + 1), _EDIM)
    offs = (jnp.arange(_NF, dtype=jnp.int32) * (_CARD + 1))[None, :]
    gidx = (x_cat.reshape(_T, _NF).astype(jnp.int32) + offs).reshape(_R)
    cat = _sc_gather(flat_tables, gidx)
    a = cat.reshape(_T, _NF * _EDIM)
    out = _tc_project(a, W, b.reshape(1, _DMODEL))
    return out.reshape(_B, _S, _DMODEL)


# speedup vs baseline: 4.9728x; 3.1964x over previous
import functools

import jax
import jax.numpy as jnp
from jax import lax
from jax.experimental import pallas as pl
from jax.experimental.pallas import tpu as pltpu
from jax.experimental.pallas import tpu_sc as plsc

_B = 4096
_S = 50
_NF = 26
_CARD = 100000
_EDIM = 32
_DMODEL = 128

_T = _B * _S
_R = _T * _NF
_LPB = 128
_CHUNK = 10

_BK = 33344
_QR = _BK // 4
_NB = 3
_PLINES = _NF * _NB * _QR


def _tc_repack(tables):
    def rk(a_ref, o_ref):
        a = a_ref[0]
        o_ref[...] = jnp.concatenate(
            [a[0:_QR], a[_QR : 2 * _QR], a[2 * _QR : 3 * _QR], a[3 * _QR :]],
            axis=1,
        )

    return pl.pallas_call(
        rk,
        grid=(_NF * _NB,),
        in_specs=[
            pl.BlockSpec((1, _BK, _EDIM), lambda g: (g // _NB, g % _NB, 0)),
        ],
        out_specs=pl.BlockSpec((_QR, _LPB), lambda g: (g, 0)),
        out_shape=jax.ShapeDtypeStruct((_PLINES, _LPB), jnp.float32),
    )(tables)


def _sc_gather(ptab32, gidx):
    info = plsc.get_sparse_core_info()
    nw = info.num_cores * info.num_subcores
    nblk = _R // _LPB
    blocks_per_w = nblk // nw
    iters = blocks_per_w // _CHUNK
    crows = _CHUNK * _LPB

    mesh = plsc.VectorSubcoreMesh(core_axis_name="c", subcore_axis_name="s")

    @functools.partial(
        pl.kernel,
        mesh=mesh,
        compiler_params=pltpu.CompilerParams(use_tc_tiling_on_sc=False),
        out_type=jax.ShapeDtypeStruct((_R, _EDIM), jnp.float32),
        scratch_types=[
            pltpu.VMEM((_CHUNK, _LPB), jnp.int32),
            pltpu.VMEM((crows, _EDIM), jnp.float32),
            pltpu.SemaphoreType.DMA,
        ],
    )
    def k(tab32, gidx_hbm, out32, idx_v, rows_v, sem):
        wid = lax.axis_index("s") * info.num_cores + lax.axis_index("c")
        base = wid * blocks_per_w

        def body(it, carry):
            blk = base + it * _CHUNK
            row0 = blk * _LPB
            pltpu.sync_copy(gidx_hbm.at[pl.ds(blk, _CHUNK)], idx_v)
            copies = []
            for j in range(_CHUNK):
                copies.append(
                    pltpu.async_copy(
                        tab32.at[idx_v.at[j]],
                        rows_v.at[pl.ds(j * _LPB, _LPB)],
                        sem,
                    )
                )
            for c in copies:
                c.wait()
            pltpu.sync_copy(rows_v, out32.at[pl.ds(row0, crows)])
            return carry

        lax.fori_loop(0, iters, body, 0)

    return k(ptab32, gidx)


def _tc_project(c3, m3, bias4):
    l4 = 512
    nt4 = _T // 4 // l4

    def mm(c_ref, m_ref, b_ref, o_ref):
        f = pl.program_id(1)
        a = c_ref[0].astype(jnp.bfloat16)
        part = jnp.dot(a, m_ref[f], preferred_element_type=jnp.float32)

        @pl.when(f == 0)
        def _():
            o_ref[...] = part + b_ref[...]

        @pl.when(f > 0)
        def _():
            o_ref[...] += part

    return pl.pallas_call(
        mm,
        grid=(nt4, _NF),
        in_specs=[
            pl.BlockSpec((1, l4, _LPB), lambda i, f: (f, i, 0)),
            pl.BlockSpec((_NF, _LPB, 4 * _DMODEL), lambda i, f: (0, 0, 0)),
            pl.BlockSpec((1, 4 * _DMODEL), lambda i, f: (0, 0)),
        ],
        out_specs=pl.BlockSpec((l4, 4 * _DMODEL), lambda i, f: (i, 0)),
        out_shape=jax.ShapeDtypeStruct((_T // 4, 4 * _DMODEL), jnp.float32),
    )(c3, m3, bias4)


def kernel(x_cat, tables, W, b):
    ptab = _tc_repack(tables)

    xt = x_cat.reshape(_T, _NF).astype(jnp.int32).T
    foff = (jnp.arange(_NF, dtype=jnp.int32) * _NB)[:, None]
    rb = xt // _BK
    ip = xt % _BK
    j = (foff + rb) * _BK + 4 * (ip % _QR) + ip // _QR
    gidx = j.reshape(_R // _LPB, _LPB)

    cat = _sc_gather(ptab.reshape(4 * _PLINES, _EDIM), gidx)
    c3 = cat.reshape(_NF, _T // 4, _LPB)

    w3 = W.reshape(_NF, _EDIM, _DMODEL).astype(jnp.bfloat16)
    eye4 = jnp.eye(4, dtype=jnp.bfloat16)
    m3 = jnp.einsum("fed,cq->fceqd", w3, eye4).reshape(
        _NF, _LPB, 4 * _DMODEL
    )
    bias4 = jnp.tile(b, 4).reshape(1, 4 * _DMODEL)

    out4 = _tc_project(c3, m3, bias4)
    return out4.reshape(_B, _S, _DMODEL)


# speedup vs baseline: 7.4237x; 1.4929x over previous
import functools

import jax
import jax.numpy as jnp
from jax import lax
from jax.experimental import pallas as pl
from jax.experimental.pallas import tpu as pltpu
from jax.experimental.pallas import tpu_sc as plsc

_B = 4096
_S = 50
_NF = 26
_CARD = 100000
_EDIM = 32
_DMODEL = 128

_T = _B * _S
_R = _T * _NF
_LPB = 128
_CHUNK = 10

_BK = 33344
_QR = _BK // 4
_NB = 3
_PLINES = _NF * _NB * _QR


def _tc_repack(tables):
    def rk(a_ref, o_ref):
        a = a_ref[0]
        o_ref[...] = jnp.concatenate(
            [a[0:_QR], a[_QR : 2 * _QR], a[2 * _QR : 3 * _QR], a[3 * _QR :]],
            axis=1,
        )

    return pl.pallas_call(
        rk,
        grid=(_NF * _NB,),
        in_specs=[
            pl.BlockSpec((1, _BK, _EDIM), lambda g: (g // _NB, g % _NB, 0)),
        ],
        out_specs=pl.BlockSpec((_QR, _LPB), lambda g: (g, 0)),
        out_shape=jax.ShapeDtypeStruct((_PLINES, _LPB), jnp.float32),
    )(tables)


def _sc_gather(ptab32, gidx):
    info = plsc.get_sparse_core_info()
    nw = info.num_cores * info.num_subcores
    nblk = _R // _LPB
    blocks_per_w = nblk // nw
    iters = blocks_per_w // _CHUNK
    crows = _CHUNK * _LPB

    mesh = plsc.VectorSubcoreMesh(core_axis_name="c", subcore_axis_name="s")

    @functools.partial(
        pl.kernel,
        mesh=mesh,
        compiler_params=pltpu.CompilerParams(use_tc_tiling_on_sc=False),
        out_type=jax.ShapeDtypeStruct((_R, _EDIM), jnp.float32),
        scratch_types=[
            pltpu.VMEM((_CHUNK, _LPB), jnp.int32),
            pltpu.VMEM((crows, _EDIM), jnp.float32),
            pltpu.SemaphoreType.DMA,
        ],
    )
    def k(tab32, gidx_hbm, out128, idx_v, rows_v, sem):
        wid = lax.axis_index("s") * info.num_cores + lax.axis_index("c")
        base = wid * blocks_per_w

        def body(it, carry):
            blk = base + it * _CHUNK
            pltpu.sync_copy(gidx_hbm.at[pl.ds(blk, _CHUNK)], idx_v)
            copies = []
            for j in range(_CHUNK):
                copies.append(
                    pltpu.async_copy(
                        tab32.at[idx_v.at[j]],
                        rows_v.at[pl.ds(j * _LPB, _LPB)],
                        sem,
                    )
                )
            for c in copies:
                c.wait()
            pltpu.sync_copy(rows_v, out128.at[pl.ds(blk * _LPB, crows)])
            return carry

        lax.fori_loop(0, iters, body, 0)

    return k(ptab32, gidx)


def _tc_project(c3, m3, bias4):
    l4 = 512
    nt4 = _T // 4 // l4

    def mm(c_ref, m_ref, b_ref, o_ref):
        acc = b_ref[...].astype(jnp.float32) + jnp.zeros(
            (l4, 4 * _DMODEL), jnp.float32
        )
        for f in range(_NF):
            acc += jnp.dot(
                c_ref[f].astype(jnp.bfloat16),
                m_ref[f],
                preferred_element_type=jnp.float32,
            )
        o_ref[...] = acc

    return pl.pallas_call(
        mm,
        grid=(nt4,),
        in_specs=[
            pl.BlockSpec((_NF, l4, _LPB), lambda i: (0, i, 0)),
            pl.BlockSpec((_NF, _LPB, 4 * _DMODEL), lambda i: (0, 0, 0)),
            pl.BlockSpec((1, 4 * _DMODEL), lambda i: (0, 0)),
        ],
        out_specs=pl.BlockSpec((l4, 4 * _DMODEL), lambda i: (i, 0)),
        out_shape=jax.ShapeDtypeStruct((_T // 4, 4 * _DMODEL), jnp.float32),
    )(c3, m3, bias4)


def kernel(x_cat, tables, W, b):
    ptab = _tc_repack(tables)

    x2 = x_cat.reshape(_T, _NF).astype(jnp.int32)
    foff = (jnp.arange(_NF, dtype=jnp.int32) * _NB)[None, :]
    rb = x2 // _BK
    ip = x2 % _BK
    j2 = (foff + rb) * _BK + 4 * (ip % _QR) + ip // _QR
    gidx = j2.T.reshape(_R // _LPB, _LPB)

    cat = _sc_gather(ptab.reshape(4 * _PLINES, _EDIM), gidx)
    c3 = cat.reshape(_NF, _T // 4, _LPB)

    w3 = W.reshape(_NF, _EDIM, _DMODEL).astype(jnp.bfloat16)
    eye4 = jnp.eye(4, dtype=jnp.bfloat16)
    m3 = jnp.einsum("fed,cq->fceqd", w3, eye4).reshape(
        _NF, _LPB, 4 * _DMODEL
    )
    bias4 = jnp.tile(b, 4).reshape(1, 4 * _DMODEL)

    out4 = _tc_project(c3, m3, bias4)
    return out4.reshape(_B, _S, _DMODEL)
